# baseline (speedup 1.0000x reference)
import functools

import jax
import jax.numpy as jnp
from jax import lax
from jax.experimental import pallas as pl
from jax.experimental.pallas import tpu as pltpu
from jax.experimental.pallas import tpu_sc as plsc

N = 10000
E = 320000
D = 128
NC = 2
NS = 16
NW = NC * NS
CH = 128
NCHUNK = 80
EPT = CH * NCHUNK
EPAD = NW * EPT
NP = 10112
RPT = NP // NS


def _seg_sum_body(feat_hbm, srcp_hbm, dstp_hbm, zeros_hbm, out_hbm,
                  agg_sh, is0, is1, id0, id1, rows0, rows1,
                  sem_i0, sem_i1, sem_r0, sem_r1):
    c = lax.axis_index("c")
    s = lax.axis_index("s")
    ebase = (c * NS + s) * EPT

    def idx_copies(i, bs, bd, sem):
        return (pltpu.make_async_copy(
                    srcp_hbm.at[pl.ds(ebase + i * CH, CH)], bs, sem),
                pltpu.make_async_copy(
                    dstp_hbm.at[pl.ds(ebase + i * CH, CH)], bd, sem))

    def fire_idx(i, bs, bd, sem):
        for cp in idx_copies(i, bs, bd, sem):
            cp.start()

    def wait_idx(i, bs, bd, sem):
        for cp in idx_copies(i, bs, bd, sem):
            cp.wait()

    def gather(bs, rows, sem):
        return pltpu.make_async_copy(feat_hbm.at[bs], rows, sem)

    fire_idx(0, is0, id0, sem_i0)
    fire_idx(1, is1, id1, sem_i1)
    pltpu.sync_copy(zeros_hbm.at[pl.ds(s * RPT, RPT)],
                    agg_sh.at[pl.ds(s * RPT, RPT)])
    wait_idx(0, is0, id0, sem_i0)
    plsc.subcore_barrier()

    @pl.loop(0, NCHUNK, step=2)
    def chunk(i):
        g0 = gather(is0, rows0, sem_r0)
        g0.start()
        g0.wait()
        pltpu.sync_copy(rows0, agg_sh.at[id0], add=True)

        @pl.when(i + 2 < NCHUNK)
        def _():
            fire_idx(i + 2, is0, id0, sem_i0)

        wait_idx(i + 1, is1, id1, sem_i1)

        g1 = gather(is1, rows1, sem_r1)
        g1.start()
        g1.wait()
        pltpu.sync_copy(rows1, agg_sh.at[id1], add=True)

        @pl.when(i + 3 < NCHUNK)
        def _():
            fire_idx(i + 3, is1, id1, sem_i1)

        @pl.when(i + 2 < NCHUNK)
        def _():
            wait_idx(i + 2, is0, id0, sem_i0)

    plsc.subcore_barrier()

    pltpu.sync_copy(agg_sh.at[pl.ds(s * RPT, RPT)],
                    out_hbm.at[pl.ds(c * NP + s * RPT, RPT)])


_seg_sum = pl.kernel(
    _seg_sum_body,
    out_type=jax.ShapeDtypeStruct((NC * NP, D), jnp.float32),
    mesh=plsc.VectorSubcoreMesh(core_axis_name="c", subcore_axis_name="s",
                                num_cores=NC, num_subcores=NS),
    scratch_types=[
        pltpu.VMEM_SHARED((NP, D), jnp.float32),
        pltpu.VMEM((CH,), jnp.int32),
        pltpu.VMEM((CH,), jnp.int32),
        pltpu.VMEM((CH,), jnp.int32),
        pltpu.VMEM((CH,), jnp.int32),
        pltpu.VMEM((CH, D), jnp.float32),
        pltpu.VMEM((CH, D), jnp.float32),
        pltpu.SemaphoreType.DMA,
        pltpu.SemaphoreType.DMA,
        pltpu.SemaphoreType.DMA,
        pltpu.SemaphoreType.DMA,
    ],
)

BM = 2000


def _mlp_body(relu, p0_ref, p1_ref, x_ref, w_ref, o_ref):
    acc = p0_ref[...] + p1_ref[...] + x_ref[...]
    y = lax.dot_general(acc, w_ref[...], (((1,), (1,)), ((), ())),
                        preferred_element_type=jnp.float32)
    o_ref[...] = jnp.maximum(y, 0.0) if relu else y


def _mlp(p0, p1, x, w, relu):
    dout = w.shape[0]
    return pl.pallas_call(
        functools.partial(_mlp_body, relu),
        grid=(N // BM,),
        in_specs=[
            pl.BlockSpec((BM, D), lambda i: (i, 0)),
            pl.BlockSpec((BM, D), lambda i: (i, 0)),
            pl.BlockSpec((BM, D), lambda i: (i, 0)),
            pl.BlockSpec((dout, D), lambda i: (0, 0)),
        ],
        out_specs=pl.BlockSpec((BM, dout), lambda i: (i, 0)),
        out_shape=jax.ShapeDtypeStruct((N, dout), jnp.float32),
    )(p0, p1, x, w)


@jax.jit
def kernel(x, edge_index, W1, W2):
    src = edge_index[0]
    dst = edge_index[1]
    pad = EPAD - E
    srcp = jnp.concatenate([src, jnp.zeros((pad,), jnp.int32)])
    dstp = jnp.concatenate([dst, jnp.full((pad,), NP - 1, jnp.int32)])
    zeros = jnp.zeros((NP, D), jnp.float32)
    p1 = _seg_sum(x, srcp, dstp, zeros)
    h = _mlp(p1[:N], p1[NP:NP + N], x, W1, relu=True)
    p2 = _seg_sum(h, srcp, dstp, zeros)
    out = _mlp(p2[:N], p2[NP:NP + N], h, W2, relu=False)
    return out


# speedup vs baseline: 1.7458x; 1.7458x over previous
import functools

import jax
import jax.numpy as jnp
from jax import lax
from jax.experimental import pallas as pl
from jax.experimental.pallas import tpu as pltpu
from jax.experimental.pallas import tpu_sc as plsc

N = 10000
E = 320000
D = 128
NC = 2
NS = 16
NW = NC * NS
CH = 80
NCHUNK = 126
EPT = CH * NCHUNK
EPAD = NW * EPT
NP = 10112
RPT = NP // NS


def _seg_sum_body(feat_hbm, srcp_hbm, dstp_hbm, zeros_hbm, out_hbm,
                  agg_sh, is0, is1, id0, id1, rows0, rows1,
                  sem_i0, sem_i1, sem_r0, sem_r1):
    c = lax.axis_index("c")
    s = lax.axis_index("s")
    ebase = (c * NS + s) * EPT

    def idx_copies(i, bs, bd, sem):
        return (pltpu.make_async_copy(
                    srcp_hbm.at[pl.ds(ebase + i * CH, CH)], bs, sem),
                pltpu.make_async_copy(
                    dstp_hbm.at[pl.ds(ebase + i * CH, CH)], bd, sem))

    def fire_idx(i, bs, bd, sem):
        for cp in idx_copies(i, bs, bd, sem):
            cp.start()

    def wait_idx(i, bs, bd, sem):
        for cp in idx_copies(i, bs, bd, sem):
            cp.wait()

    def gather(bs, rows, sem):
        return pltpu.make_async_copy(feat_hbm.at[bs], rows, sem)

    fire_idx(0, is0, id0, sem_i0)
    fire_idx(1, is1, id1, sem_i1)
    pltpu.sync_copy(zeros_hbm.at[pl.ds(s * RPT, RPT)],
                    agg_sh.at[pl.ds(s * RPT, RPT)])
    wait_idx(0, is0, id0, sem_i0)
    plsc.subcore_barrier()

    @pl.loop(0, NCHUNK, step=2)
    def chunk(i):
        g0 = gather(is0, rows0, sem_r0)
        g0.start()
        g0.wait()
        pltpu.sync_copy(rows0, agg_sh.at[id0], add=True)

        @pl.when(i + 2 < NCHUNK)
        def _():
            fire_idx(i + 2, is0, id0, sem_i0)

        wait_idx(i + 1, is1, id1, sem_i1)

        g1 = gather(is1, rows1, sem_r1)
        g1.start()
        g1.wait()
        pltpu.sync_copy(rows1, agg_sh.at[id1], add=True)

        @pl.when(i + 3 < NCHUNK)
        def _():
            fire_idx(i + 3, is1, id1, sem_i1)

        @pl.when(i + 2 < NCHUNK)
        def _():
            wait_idx(i + 2, is0, id0, sem_i0)

    plsc.subcore_barrier()

    pltpu.sync_copy(agg_sh.at[pl.ds(s * RPT, RPT)],
                    out_hbm.at[pl.ds(c * NP + s * RPT, RPT)])


_seg_sum = pl.kernel(
    _seg_sum_body,
    out_type=jax.ShapeDtypeStruct((NC * NP, D), jnp.float32),
    mesh=plsc.VectorSubcoreMesh(core_axis_name="c", subcore_axis_name="s",
                                num_cores=NC, num_subcores=NS),
    scratch_types=[
        pltpu.VMEM_SHARED((NP, D), jnp.float32),
        pltpu.VMEM((CH,), jnp.int32),
        pltpu.VMEM((CH,), jnp.int32),
        pltpu.VMEM((CH,), jnp.int32),
        pltpu.VMEM((CH,), jnp.int32),
        pltpu.VMEM((CH, D), jnp.float32),
        pltpu.VMEM((CH, D), jnp.float32),
        pltpu.SemaphoreType.DMA,
        pltpu.SemaphoreType.DMA,
        pltpu.SemaphoreType.DMA,
        pltpu.SemaphoreType.DMA,
    ],
)

BM = 2000


def _mlp_body(relu, p0_ref, p1_ref, x_ref, w_ref, o_ref):
    acc = p0_ref[...] + p1_ref[...] + x_ref[...]
    y = lax.dot_general(acc, w_ref[...], (((1,), (1,)), ((), ())),
                        preferred_element_type=jnp.float32)
    o_ref[...] = jnp.maximum(y, 0.0) if relu else y


def _mlp(p0, p1, x, w, relu):
    dout = w.shape[0]
    return pl.pallas_call(
        functools.partial(_mlp_body, relu),
        grid=(N // BM,),
        in_specs=[
            pl.BlockSpec((BM, D), lambda i: (i, 0)),
            pl.BlockSpec((BM, D), lambda i: (i, 0)),
            pl.BlockSpec((BM, D), lambda i: (i, 0)),
            pl.BlockSpec((dout, D), lambda i: (0, 0)),
        ],
        out_specs=pl.BlockSpec((BM, dout), lambda i: (i, 0)),
        out_shape=jax.ShapeDtypeStruct((N, dout), jnp.float32),
    )(p0, p1, x, w)


@jax.jit
def kernel(x, edge_index, W1, W2):
    src = edge_index[0]
    dst = edge_index[1]
    pad = EPAD - E
    srcp = jnp.concatenate([src, jnp.zeros((pad,), jnp.int32)])
    dstp = jnp.concatenate([dst, jnp.full((pad,), NP - 1, jnp.int32)])
    zeros = jnp.zeros((NP, D), jnp.float32)
    p1 = _seg_sum(x, srcp, dstp, zeros)
    h = _mlp(p1[:N], p1[NP:NP + N], x, W1, relu=True)
    p2 = _seg_sum(h, srcp, dstp, zeros)
    out = _mlp(p2[:N], p2[NP:NP + N], h, W2, relu=False)
    return out


# speedup vs baseline: 2.1731x; 1.2448x over previous
import functools

import jax
import jax.numpy as jnp
from jax import lax
from jax.experimental import pallas as pl
from jax.experimental.pallas import tpu as pltpu
from jax.experimental.pallas import tpu_sc as plsc

N = 10000
E = 320000
D = 128
NC = 2
NS = 16
NW = NC * NS
CH = 80
NCHUNK = 126
EPT = CH * NCHUNK
EPAD = NW * EPT
NP = 10112
RPT = NP // NS


def _seg_sum_body(feat_hbm, srcp_hbm, dstp_hbm, zeros_hbm, out_hbm,
                  agg_sh, is0, is1, id0, id1, rows0, rows1,
                  sem_i0, sem_i1, sem_r0, sem_r1):
    c = lax.axis_index("c")
    s = lax.axis_index("s")
    ebase = (c * NS + s) * EPT

    def idx_copies(i, bs, bd, sem):
        return (pltpu.make_async_copy(
                    srcp_hbm.at[pl.ds(ebase + i * CH, CH)], bs, sem),
                pltpu.make_async_copy(
                    dstp_hbm.at[pl.ds(ebase + i * CH, CH)], bd, sem))

    def fire_idx(i, bs, bd, sem):
        for cp in idx_copies(i, bs, bd, sem):
            cp.start()

    def wait_idx(i, bs, bd, sem):
        for cp in idx_copies(i, bs, bd, sem):
            cp.wait()

    def gather(bs, rows, sem):
        return pltpu.make_async_copy(feat_hbm.at[bs], rows, sem)

    fire_idx(0, is0, id0, sem_i0)
    fire_idx(1, is1, id1, sem_i1)
    pltpu.sync_copy(zeros_hbm.at[pl.ds(s * RPT, RPT)],
                    agg_sh.at[pl.ds(s * RPT, RPT)])
    wait_idx(0, is0, id0, sem_i0)
    gather(is0, rows0, sem_r0).start()
    plsc.subcore_barrier()

    @pl.loop(0, NCHUNK, step=2)
    def chunk(i):
        wait_idx(i + 1, is1, id1, sem_i1)
        gather(is1, rows1, sem_r1).start()
        gather(is0, rows0, sem_r0).wait()
        pltpu.sync_copy(rows0, agg_sh.at[id0], add=True)

        @pl.when(i + 2 < NCHUNK)
        def _():
            fire_idx(i + 2, is0, id0, sem_i0)
            wait_idx(i + 2, is0, id0, sem_i0)
            gather(is0, rows0, sem_r0).start()

        gather(is1, rows1, sem_r1).wait()
        pltpu.sync_copy(rows1, agg_sh.at[id1], add=True)

        @pl.when(i + 3 < NCHUNK)
        def _():
            fire_idx(i + 3, is1, id1, sem_i1)

    plsc.subcore_barrier()

    pltpu.sync_copy(agg_sh.at[pl.ds(s * RPT, RPT)],
                    out_hbm.at[pl.ds(c * NP + s * RPT, RPT)])


_seg_sum = pl.kernel(
    _seg_sum_body,
    out_type=jax.ShapeDtypeStruct((NC * NP, D), jnp.float32),
    mesh=plsc.VectorSubcoreMesh(core_axis_name="c", subcore_axis_name="s",
                                num_cores=NC, num_subcores=NS),
    scratch_types=[
        pltpu.VMEM_SHARED((NP, D), jnp.float32),
        pltpu.VMEM((CH,), jnp.int32),
        pltpu.VMEM((CH,), jnp.int32),
        pltpu.VMEM((CH,), jnp.int32),
        pltpu.VMEM((CH,), jnp.int32),
        pltpu.VMEM((CH, D), jnp.float32),
        pltpu.VMEM((CH, D), jnp.float32),
        pltpu.SemaphoreType.DMA,
        pltpu.SemaphoreType.DMA,
        pltpu.SemaphoreType.DMA,
        pltpu.SemaphoreType.DMA,
    ],
)

BM = 2000


def _mlp_body(relu, p0_ref, p1_ref, x_ref, w_ref, o_ref):
    acc = p0_ref[...] + p1_ref[...] + x_ref[...]
    y = lax.dot_general(acc, w_ref[...], (((1,), (1,)), ((), ())),
                        preferred_element_type=jnp.float32)
    o_ref[...] = jnp.maximum(y, 0.0) if relu else y


def _mlp(p0, p1, x, w, relu):
    dout = w.shape[0]
    return pl.pallas_call(
        functools.partial(_mlp_body, relu),
        grid=(N // BM,),
        in_specs=[
            pl.BlockSpec((BM, D), lambda i: (i, 0)),
            pl.BlockSpec((BM, D), lambda i: (i, 0)),
            pl.BlockSpec((BM, D), lambda i: (i, 0)),
            pl.BlockSpec((dout, D), lambda i: (0, 0)),
        ],
        out_specs=pl.BlockSpec((BM, dout), lambda i: (i, 0)),
        out_shape=jax.ShapeDtypeStruct((N, dout), jnp.float32),
    )(p0, p1, x, w)


@jax.jit
def kernel(x, edge_index, W1, W2):
    src = edge_index[0]
    dst = edge_index[1]
    pad = EPAD - E
    srcp = jnp.concatenate([src, jnp.zeros((pad,), jnp.int32)])
    dstp = jnp.concatenate([dst, jnp.full((pad,), NP - 1, jnp.int32)])
    zeros = jnp.zeros((NP, D), jnp.float32)
    p1 = _seg_sum(x, srcp, dstp, zeros)
    h = _mlp(p1[:N], p1[NP:NP + N], x, W1, relu=True)
    p2 = _seg_sum(h, srcp, dstp, zeros)
    out = _mlp(p2[:N], p2[NP:NP + N], h, W2, relu=False)
    return out
